# baseline (device time: 44650 ns/iter reference)
import math

import jax
import jax.numpy as jnp
from jax import lax
from jax.experimental import pallas as pl
from jax.experimental.pallas import tpu as pltpu

N_DEV = 4
B, SQ, D = 8, 128, 512
HQ, DH = 16, 64
H_LOC = HQ // N_DEV
B_LOC = B // N_DEV
R = B * SQ
R_LOC = B_LOC * SQ
HD_LOC = H_LOC * DH


def kernel(x, Wq, Wk, Wv, Wo):
    def body(x_ref, wq_ref, wk_ref, wv_ref, wo_ref, out_ref,
             xfull, ctx_ref, p_ref, rs_ref,
             xs_send, xs_recv, ps_send, ps_recv):
        me = lax.axis_index("i")

        bsem = pltpu.get_barrier_semaphore()
        for k in range(1, N_DEV):
            pl.semaphore_signal(
                bsem, inc=1,
                device_id=((me + k) % N_DEV,),
                device_id_type=pl.DeviceIdType.MESH,
            )
        pl.semaphore_wait(bsem, N_DEV - 1)

        my_rows = pl.ds(me * R_LOC, R_LOC)
        xfull[my_rows, :] = x_ref[...].astype(jnp.bfloat16).reshape(R_LOC, D)
        x_sends = []
        for k in range(1, N_DEV):
            t = (me + k) % N_DEV
            rdma = pltpu.make_async_remote_copy(
                src_ref=xfull.at[my_rows, :],
                dst_ref=xfull.at[my_rows, :],
                send_sem=xs_send.at[k - 1],
                recv_sem=xs_recv.at[k - 1],
                device_id=(t,),
                device_id_type=pl.DeviceIdType.MESH,
            )
            rdma.start()
            x_sends.append(rdma)
        for k in range(1, N_DEV):
            s = (me - k) % N_DEV
            src_rows = pl.ds(s * R_LOC, R_LOC)
            recv = pltpu.make_async_remote_copy(
                src_ref=xfull.at[src_rows, :],
                dst_ref=xfull.at[src_rows, :],
                send_sem=xs_send.at[k - 1],
                recv_sem=xs_recv.at[k - 1],
                device_id=(s,),
                device_id_type=pl.DeviceIdType.MESH,
            )
            recv.wait_recv()

        xf = xfull[...]
        qf = jnp.dot(xf, wq_ref[...].astype(jnp.bfloat16),
                     preferred_element_type=jnp.float32)
        kf = jnp.dot(xf, wk_ref[...].astype(jnp.bfloat16),
                     preferred_element_type=jnp.float32)
        vf = jnp.dot(xf, wv_ref[...].astype(jnp.bfloat16),
                     preferred_element_type=jnp.float32)

        row = lax.broadcasted_iota(jnp.int32, (R, HD_LOC), 0)
        col = lax.broadcasted_iota(jnp.int32, (R, HD_LOC), 1)
        pos = (row % SQ).astype(jnp.float32)
        dpair = (((col % DH) // 2) * 2).astype(jnp.float32)
        invf = jnp.exp(dpair * (-math.log(10000.0) / DH))
        ang = pos * invf
        cosv = jnp.cos(ang)
        sinv = jnp.sin(ang)
        even = (col % 2) == 0

        def rope(t):
            tm1 = jnp.concatenate([t[:, 1:], t[:, :1]], axis=1)
            tp1 = jnp.concatenate([t[:, -1:], t[:, :-1]], axis=1)
            tr = jnp.where(even, -tm1, tp1)
            return t * cosv + tr * sinv

        qr = rope(qf).astype(jnp.bfloat16)
        kr = rope(kf).astype(jnp.bfloat16)
        vb = vf.astype(jnp.bfloat16)

        for b in range(B):
            rs = slice(b * SQ, (b + 1) * SQ)
            for h in range(H_LOC):
                cs = slice(h * DH, (h + 1) * DH)
                s = lax.dot_general(
                    qr[rs, cs], kr[rs, cs],
                    (((1,), (1,)), ((), ())),
                    preferred_element_type=jnp.float32,
                ) * 0.125
                m = jnp.max(s, axis=-1, keepdims=True)
                w = jnp.exp(s - m)
                w = w / jnp.sum(w, axis=-1, keepdims=True)
                c = jnp.dot(w.astype(jnp.bfloat16), vb[rs, cs],
                            preferred_element_type=jnp.float32)
                ctx_ref[rs, cs] = c.astype(jnp.bfloat16)

        p_ref[...] = jnp.dot(ctx_ref[...], wo_ref[...].astype(jnp.bfloat16),
                             preferred_element_type=jnp.float32)

        p_sends = []
        for k in range(1, N_DEV):
            t = (me + k) % N_DEV
            rdma = pltpu.make_async_remote_copy(
                src_ref=p_ref.at[pl.ds(t * R_LOC, R_LOC), :],
                dst_ref=rs_ref.at[k - 1],
                send_sem=ps_send.at[k - 1],
                recv_sem=ps_recv.at[k - 1],
                device_id=(t,),
                device_id_type=pl.DeviceIdType.MESH,
            )
            rdma.start()
            p_sends.append(rdma)
        acc = p_ref[my_rows, :]
        for k in range(1, N_DEV):
            s = (me - k) % N_DEV
            recv = pltpu.make_async_remote_copy(
                src_ref=rs_ref.at[k - 1],
                dst_ref=rs_ref.at[k - 1],
                send_sem=ps_send.at[k - 1],
                recv_sem=ps_recv.at[k - 1],
                device_id=(s,),
                device_id_type=pl.DeviceIdType.MESH,
            )
            recv.wait_recv()
            acc = acc + rs_ref[k - 1]
        for rdma in x_sends + p_sends:
            rdma.wait_send()

        out_ref[...] = acc.reshape(B_LOC, SQ, D)

    return pl.pallas_call(
        body,
        out_shape=jax.ShapeDtypeStruct((B_LOC, SQ, D), jnp.float32),
        in_specs=[pl.BlockSpec(memory_space=pltpu.VMEM)] * 5,
        out_specs=pl.BlockSpec(memory_space=pltpu.VMEM),
        scratch_shapes=[
            pltpu.VMEM((R, D), jnp.bfloat16),
            pltpu.VMEM((R, HD_LOC), jnp.bfloat16),
            pltpu.VMEM((R, D), jnp.float32),
            pltpu.VMEM((N_DEV - 1, R_LOC, D), jnp.float32),
            pltpu.SemaphoreType.DMA((N_DEV - 1,)),
            pltpu.SemaphoreType.DMA((N_DEV - 1,)),
            pltpu.SemaphoreType.DMA((N_DEV - 1,)),
            pltpu.SemaphoreType.DMA((N_DEV - 1,)),
        ],
        compiler_params=pltpu.CompilerParams(collective_id=0),
    )(x, Wq, Wk, Wv, Wo)


# device time: 27929 ns/iter; 1.5987x vs baseline; 1.5987x over previous
import math

import jax
import jax.numpy as jnp
from jax import lax
from jax.experimental import pallas as pl
from jax.experimental.pallas import tpu as pltpu

N_DEV = 4
B, SQ, D = 8, 128, 512
HQ, DH = 16, 64
H_LOC = HQ // N_DEV
B_LOC = B // N_DEV
R_LOC = B_LOC * SQ
HD_LOC = H_LOC * DH


def kernel(x, Wq, Wk, Wv, Wo):
    def body(x_ref, wq_ref, wk_ref, wv_ref, wo_ref, out_ref,
             xfull, ctx_ref, psend_ref, rs_ref,
             xs_send, xs_recv, ps_send, ps_recv):
        me = lax.axis_index("i")

        bsem = pltpu.get_barrier_semaphore()
        for k in range(1, N_DEV):
            pl.semaphore_signal(
                bsem, inc=1,
                device_id=((me + k) % N_DEV,),
                device_id_type=pl.DeviceIdType.MESH,
            )
        pl.semaphore_wait(bsem, N_DEV - 1)

        my_rows = pl.ds(me * R_LOC, R_LOC)
        xfull[my_rows, :] = x_ref[...].astype(jnp.bfloat16).reshape(R_LOC, D)
        x_sends = []
        for k in range(1, N_DEV):
            t = (me + k) % N_DEV
            rdma = pltpu.make_async_remote_copy(
                src_ref=xfull.at[my_rows, :],
                dst_ref=xfull.at[my_rows, :],
                send_sem=xs_send.at[k - 1],
                recv_sem=xs_recv.at[k - 1],
                device_id=(t,),
                device_id_type=pl.DeviceIdType.MESH,
            )
            rdma.start()
            x_sends.append(rdma)

        wqb = wq_ref[...].astype(jnp.bfloat16)
        wkb = wk_ref[...].astype(jnp.bfloat16)
        wvb = wv_ref[...].astype(jnp.bfloat16)
        wob = wo_ref[...].astype(jnp.bfloat16)

        row = lax.broadcasted_iota(jnp.int32, (R_LOC, HD_LOC), 0)
        col = lax.broadcasted_iota(jnp.int32, (R_LOC, HD_LOC), 1)
        pos = (row % SQ).astype(jnp.float32)
        dpair = (((col % DH) // 2) * 2).astype(jnp.float32)
        invf = jnp.exp(dpair * (-math.log(10000.0) / DH))
        ang = pos * invf
        cosv = jnp.cos(ang)
        sinv = jnp.sin(ang)
        even = (col % 2) == 0

        def rope(t):
            tm1 = jnp.concatenate([t[:, 1:], t[:, :1]], axis=1)
            tp1 = jnp.concatenate([t[:, -1:], t[:, :-1]], axis=1)
            tr = jnp.where(even, -tm1, tp1)
            return t * cosv + tr * sinv

        def compute_group(g):
            xg = xfull[pl.ds(g * R_LOC, R_LOC), :]
            qg = rope(jnp.dot(xg, wqb,
                              preferred_element_type=jnp.float32))
            kg = rope(jnp.dot(xg, wkb,
                              preferred_element_type=jnp.float32))
            vg = jnp.dot(xg, wvb,
                         preferred_element_type=jnp.float32)
            qg = qg.astype(jnp.bfloat16)
            kg = kg.astype(jnp.bfloat16)
            vg = vg.astype(jnp.bfloat16)
            for b in range(B_LOC):
                rows = slice(b * SQ, (b + 1) * SQ)
                for h in range(H_LOC):
                    cs = slice(h * DH, (h + 1) * DH)
                    s = lax.dot_general(
                        qg[rows, cs], kg[rows, cs],
                        (((1,), (1,)), ((), ())),
                        preferred_element_type=jnp.float32,
                    ) * 0.125
                    m = jnp.max(s, axis=-1, keepdims=True)
                    w = jnp.exp(s - m)
                    w = w / jnp.sum(w, axis=-1, keepdims=True)
                    c = jnp.dot(w.astype(jnp.bfloat16), vg[rows, cs],
                                preferred_element_type=jnp.float32)
                    ctx_ref[rows, cs] = c.astype(jnp.bfloat16)
            return jnp.dot(ctx_ref[...], wob,
                           preferred_element_type=jnp.float32)

        acc = compute_group(me)

        p_sends = []
        for k in range(1, N_DEV):
            s = (me - k) % N_DEV
            src_rows = pl.ds(s * R_LOC, R_LOC)
            recv = pltpu.make_async_remote_copy(
                src_ref=xfull.at[src_rows, :],
                dst_ref=xfull.at[src_rows, :],
                send_sem=xs_send.at[k - 1],
                recv_sem=xs_recv.at[k - 1],
                device_id=(s,),
                device_id_type=pl.DeviceIdType.MESH,
            )
            recv.wait_recv()
            pg = compute_group(s)
            j = 3 - k
            psend_ref[j] = pg.astype(jnp.bfloat16)
            rdma = pltpu.make_async_remote_copy(
                src_ref=psend_ref.at[j],
                dst_ref=rs_ref.at[j],
                send_sem=ps_send.at[j],
                recv_sem=ps_recv.at[j],
                device_id=(s,),
                device_id_type=pl.DeviceIdType.MESH,
            )
            rdma.start()
            p_sends.append(rdma)

        for j in (2, 1, 0):
            recv = pltpu.make_async_remote_copy(
                src_ref=rs_ref.at[j],
                dst_ref=rs_ref.at[j],
                send_sem=ps_send.at[j],
                recv_sem=ps_recv.at[j],
                device_id=((me + N_DEV - 1 - j) % N_DEV,),
                device_id_type=pl.DeviceIdType.MESH,
            )
            recv.wait_recv()
            acc = acc + rs_ref[j].astype(jnp.float32)
        for rdma in x_sends + p_sends:
            rdma.wait_send()

        out_ref[...] = acc.reshape(B_LOC, SQ, D)

    return pl.pallas_call(
        body,
        out_shape=jax.ShapeDtypeStruct((B_LOC, SQ, D), jnp.float32),
        in_specs=[pl.BlockSpec(memory_space=pltpu.VMEM)] * 5,
        out_specs=pl.BlockSpec(memory_space=pltpu.VMEM),
        scratch_shapes=[
            pltpu.VMEM((N_DEV * R_LOC, D), jnp.bfloat16),
            pltpu.VMEM((R_LOC, HD_LOC), jnp.bfloat16),
            pltpu.VMEM((N_DEV - 1, R_LOC, D), jnp.bfloat16),
            pltpu.VMEM((N_DEV - 1, R_LOC, D), jnp.bfloat16),
            pltpu.SemaphoreType.DMA((N_DEV - 1,)),
            pltpu.SemaphoreType.DMA((N_DEV - 1,)),
            pltpu.SemaphoreType.DMA((N_DEV - 1,)),
            pltpu.SemaphoreType.DMA((N_DEV - 1,)),
        ],
        compiler_params=pltpu.CompilerParams(collective_id=0),
    )(x, Wq, Wk, Wv, Wo)


# device time: 26611 ns/iter; 1.6779x vs baseline; 1.0495x over previous
import math

import jax
import jax.numpy as jnp
from jax import lax
from jax.experimental import pallas as pl
from jax.experimental.pallas import tpu as pltpu

N_DEV = 4
B, SQ, D = 8, 128, 512
HQ, DH = 16, 64
H_LOC = HQ // N_DEV
B_LOC = B // N_DEV
R_LOC = B_LOC * SQ
HD_LOC = H_LOC * DH


def kernel(x, Wq, Wk, Wv, Wo):
    def body(x_ref, wq_ref, wk_ref, wv_ref, wo_ref, out_ref,
             xfull, ctx_ref, psend_ref, rs_ref,
             xs_send, xs_recv, ps_send, ps_recv):
        me = lax.axis_index("i")

        bsem = pltpu.get_barrier_semaphore()
        for k in range(1, N_DEV):
            pl.semaphore_signal(
                bsem, inc=1,
                device_id=((me + k) % N_DEV,),
                device_id_type=pl.DeviceIdType.MESH,
            )
        pl.semaphore_wait(bsem, N_DEV - 1)

        my_rows = pl.ds(me * R_LOC, R_LOC)
        xfull[my_rows, :] = x_ref[...].astype(jnp.bfloat16).reshape(R_LOC, D)
        x_sends = []
        for k in range(1, N_DEV):
            t = (me + k) % N_DEV
            rdma = pltpu.make_async_remote_copy(
                src_ref=xfull.at[my_rows, :],
                dst_ref=xfull.at[my_rows, :],
                send_sem=xs_send.at[k - 1],
                recv_sem=xs_recv.at[k - 1],
                device_id=(t,),
                device_id_type=pl.DeviceIdType.MESH,
            )
            rdma.start()
            x_sends.append(rdma)

        wqkv = jnp.concatenate(
            [wq_ref[...].astype(jnp.bfloat16),
             wk_ref[...].astype(jnp.bfloat16),
             wv_ref[...].astype(jnp.bfloat16)], axis=1)
        wob = wo_ref[...].astype(jnp.bfloat16)

        row = lax.broadcasted_iota(jnp.int32, (R_LOC, HD_LOC), 0)
        col = lax.broadcasted_iota(jnp.int32, (R_LOC, HD_LOC), 1)
        pos = (row % SQ).astype(jnp.float32)
        dpair = (((col % DH) // 2) * 2).astype(jnp.float32)
        invf = jnp.exp(dpair * (-math.log(10000.0) / DH))
        ang = pos * invf
        cosv = jnp.cos(ang)
        sinv = jnp.sin(ang)
        even = (col % 2) == 0

        def rope(t):
            tm1 = jnp.concatenate([t[:, 1:], t[:, :1]], axis=1)
            tp1 = jnp.concatenate([t[:, -1:], t[:, :-1]], axis=1)
            tr = jnp.where(even, -tm1, tp1)
            return t * cosv + tr * sinv

        brow = lax.broadcasted_iota(jnp.int32, (R_LOC, R_LOC), 0) // SQ
        bcol = lax.broadcasted_iota(jnp.int32, (R_LOC, R_LOC), 1) // SQ
        bmask = (brow == bcol).astype(jnp.float32)

        def compute_group(g):
            xg = xfull[pl.ds(g * R_LOC, R_LOC), :]
            qkv = jnp.dot(xg, wqkv,
                          preferred_element_type=jnp.float32)
            qg = (rope(qkv[:, :HD_LOC]) * 0.125).astype(jnp.bfloat16)
            kg = rope(qkv[:, HD_LOC:2 * HD_LOC]).astype(jnp.bfloat16)
            vg = qkv[:, 2 * HD_LOC:].astype(jnp.bfloat16)
            for h in range(H_LOC):
                cs = slice(h * DH, (h + 1) * DH)
                s = lax.dot_general(
                    qg[:, cs], kg[:, cs],
                    (((1,), (1,)), ((), ())),
                    preferred_element_type=jnp.float32,
                )
                w = jnp.exp(s) * bmask
                w = w / jnp.sum(w, axis=-1, keepdims=True)
                c = jnp.dot(w.astype(jnp.bfloat16), vg[:, cs],
                            preferred_element_type=jnp.float32)
                ctx_ref[:, cs] = c.astype(jnp.bfloat16)
            return jnp.dot(ctx_ref[...], wob,
                           preferred_element_type=jnp.float32)

        acc = compute_group(me)

        p_sends = []
        for k in range(1, N_DEV):
            s = (me - k) % N_DEV
            src_rows = pl.ds(s * R_LOC, R_LOC)
            recv = pltpu.make_async_remote_copy(
                src_ref=xfull.at[src_rows, :],
                dst_ref=xfull.at[src_rows, :],
                send_sem=xs_send.at[k - 1],
                recv_sem=xs_recv.at[k - 1],
                device_id=(s,),
                device_id_type=pl.DeviceIdType.MESH,
            )
            recv.wait_recv()
            pg = compute_group(s)
            j = 3 - k
            psend_ref[j] = pg.astype(jnp.bfloat16)
            rdma = pltpu.make_async_remote_copy(
                src_ref=psend_ref.at[j],
                dst_ref=rs_ref.at[j],
                send_sem=ps_send.at[j],
                recv_sem=ps_recv.at[j],
                device_id=(s,),
                device_id_type=pl.DeviceIdType.MESH,
            )
            rdma.start()
            p_sends.append(rdma)

        for j in (2, 1, 0):
            recv = pltpu.make_async_remote_copy(
                src_ref=rs_ref.at[j],
                dst_ref=rs_ref.at[j],
                send_sem=ps_send.at[j],
                recv_sem=ps_recv.at[j],
                device_id=((me + N_DEV - 1 - j) % N_DEV,),
                device_id_type=pl.DeviceIdType.MESH,
            )
            recv.wait_recv()
            acc = acc + rs_ref[j].astype(jnp.float32)
        for rdma in x_sends + p_sends:
            rdma.wait_send()

        out_ref[...] = acc.reshape(B_LOC, SQ, D)

    return pl.pallas_call(
        body,
        out_shape=jax.ShapeDtypeStruct((B_LOC, SQ, D), jnp.float32),
        in_specs=[pl.BlockSpec(memory_space=pltpu.VMEM)] * 5,
        out_specs=pl.BlockSpec(memory_space=pltpu.VMEM),
        scratch_shapes=[
            pltpu.VMEM((N_DEV * R_LOC, D), jnp.bfloat16),
            pltpu.VMEM((R_LOC, HD_LOC), jnp.bfloat16),
            pltpu.VMEM((N_DEV - 1, R_LOC, D), jnp.bfloat16),
            pltpu.VMEM((N_DEV - 1, R_LOC, D), jnp.bfloat16),
            pltpu.SemaphoreType.DMA((N_DEV - 1,)),
            pltpu.SemaphoreType.DMA((N_DEV - 1,)),
            pltpu.SemaphoreType.DMA((N_DEV - 1,)),
            pltpu.SemaphoreType.DMA((N_DEV - 1,)),
        ],
        compiler_params=pltpu.CompilerParams(collective_id=0),
    )(x, Wq, Wk, Wv, Wo)
